# Initial kernel scaffold; baseline (speedup 1.0000x reference)
import jax
import jax.numpy as jnp
from jax.experimental import pallas as pl


def kernel(x, edge_index, W):
    raise NotImplementedError("write your pallas kernel here")


# speedup vs baseline: 11.4564x; 11.4564x over previous
import functools

import jax
import jax.numpy as jnp
from jax import lax
from jax.experimental import pallas as pl
from jax.experimental.pallas import tpu as pltpu
from jax.experimental.pallas import tpu_sc as plsc

NC = 2
NS = 16
NW = NC * NS
L = 16
CHUNK = 128
TCB = 128


def _round_up(v, m):
    return (v + m - 1) // m * m


def _sc_degree(dst3, n_pad):
    nch = dst3.shape[1]
    per = n_pad // NS
    mesh = plsc.VectorSubcoreMesh(core_axis_name="c", subcore_axis_name="s")

    @functools.partial(
        pl.kernel,
        out_type=jax.ShapeDtypeStruct((NC, NS, per), jnp.float32),
        mesh=mesh,
        scratch_types=[
            pltpu.VMEM((nch, CHUNK), jnp.int32),
            pltpu.VMEM((CHUNK,), jnp.float32),
            pltpu.VMEM((per,), jnp.float32),
            pltpu.VMEM_SHARED((n_pad,), jnp.float32),
        ],
    )
    def deg_kernel(dst_hbm, out_hbm, idx_v, ones_v, zeros_v, deg_s):
        c = lax.axis_index("c")
        s = lax.axis_index("s")
        wid = c * NS + s
        pltpu.sync_copy(dst_hbm.at[wid], idx_v)
        for k in range(CHUNK // L):
            ones_v[pl.ds(k * L, L)] = jnp.ones((L,), jnp.float32)
        for k in range(per // L):
            zeros_v[pl.ds(k * L, L)] = jnp.zeros((L,), jnp.float32)
        pltpu.sync_copy(zeros_v, deg_s.at[pl.ds(s * per, per)])
        plsc.subcore_barrier()

        def body(j, carry):
            pltpu.sync_copy(ones_v, deg_s.at[idx_v.at[j]], add=True)
            return carry

        lax.fori_loop(0, nch, body, 0)
        plsc.subcore_barrier()
        pltpu.sync_copy(deg_s.at[pl.ds(s * per, per)], out_hbm.at[c, s])

    return deg_kernel(dst3)


def _sc_aggregate(xs_pad, src3, dst3, n_pad):
    h = xs_pad.shape[1]
    nch = src3.shape[1]
    per = n_pad // NS
    mesh = plsc.VectorSubcoreMesh(core_axis_name="c", subcore_axis_name="s")

    @functools.partial(
        pl.kernel,
        out_type=jax.ShapeDtypeStruct((NC, NS, per, h), jnp.float32),
        mesh=mesh,
        scratch_types=[
            pltpu.VMEM((CHUNK, h), jnp.float32),
            pltpu.VMEM((CHUNK, h), jnp.float32),
            pltpu.VMEM((CHUNK,), jnp.int32),
            pltpu.VMEM((CHUNK,), jnp.int32),
            pltpu.VMEM((CHUNK,), jnp.int32),
            pltpu.VMEM((CHUNK,), jnp.int32),
            pltpu.VMEM_SHARED((n_pad, h), jnp.float32),
            pltpu.SemaphoreType.DMA,
            pltpu.SemaphoreType.DMA,
            pltpu.SemaphoreType.DMA,
            pltpu.SemaphoreType.DMA,
            pltpu.SemaphoreType.DMA,
            pltpu.SemaphoreType.DMA,
        ],
    )
    def agg_kernel(xs_hbm, src_hbm, dst_hbm, out_hbm,
                   r0, r1, si0, si1, di0, di1, acc_s,
                   gs0, gs1, is0, is1, id0, id1):
        c = lax.axis_index("c")
        s = lax.axis_index("s")
        wid = c * NS + s
        rbuf = (r0, r1)
        sibuf = (si0, si1)
        dibuf = (di0, di1)
        gsem = (gs0, gs1)
        isem = (is0, is1)
        dsem = (id0, id1)

        def start_idx(k, p):
            pltpu.async_copy(src_hbm.at[wid, k], sibuf[p], isem[p])
            pltpu.async_copy(dst_hbm.at[wid, k], dibuf[p], dsem[p])

        def wait_idx(p):
            pltpu.make_async_copy(src_hbm.at[wid, 0], sibuf[p], isem[p]).wait()
            pltpu.make_async_copy(dst_hbm.at[wid, 0], dibuf[p], dsem[p]).wait()

        def start_gather(p):
            pltpu.async_copy(xs_hbm.at[sibuf[p]], rbuf[p], gsem[p])

        def wait_gather(p):
            pltpu.make_async_copy(xs_hbm.at[sibuf[p]], rbuf[p], gsem[p]).wait()

        zero = jnp.zeros((L,), jnp.float32)

        def zrow(i, carry):
            for k in range(h // L):
                r0[i, pl.ds(k * L, L)] = zero
            return carry

        lax.fori_loop(0, CHUNK, zrow, 0)
        for b in range(per // CHUNK):
            pltpu.sync_copy(r0, acc_s.at[pl.ds(s * per + b * CHUNK, CHUNK)])
        plsc.subcore_barrier()

        start_idx(0, 0)
        start_idx(1, 1)
        wait_idx(0)
        start_gather(0)

        def step(k, p):
            q = 1 - p
            wait_idx(q)
            start_gather(q)
            wait_gather(p)
            pltpu.sync_copy(rbuf[p], acc_s.at[dibuf[p]], add=True)
            start_idx(k + 2, p)

        def pair(i, carry):
            step(2 * i, 0)
            step(2 * i + 1, 1)
            return carry

        lax.fori_loop(0, (nch - 2) // 2, pair, 0)
        wait_idx(1)
        start_gather(1)
        wait_gather(0)
        pltpu.sync_copy(r0, acc_s.at[di0], add=True)
        wait_gather(1)
        pltpu.sync_copy(r1, acc_s.at[di1], add=True)

        plsc.subcore_barrier()
        pltpu.sync_copy(acc_s.at[pl.ds(s * per, per)], out_hbm.at[c, s])

    return agg_kernel(xs_pad, src3, dst3)


def _transform_body(x_ref, w_ref, d_ref, o_ref):
    dtot = d_ref[:, 0:1] + d_ref[:, 1:2] + 1.0
    dinv = lax.rsqrt(dtot)
    xw = jnp.dot(x_ref[...], w_ref[...], preferred_element_type=jnp.float32,
                 precision=lax.Precision.HIGHEST)
    o_ref[...] = xw * dinv


def _tc_transform(x_pad, w, deg_t):
    n_pad, din = x_pad.shape
    h = w.shape[1]
    return pl.pallas_call(
        _transform_body,
        grid=(n_pad // TCB,),
        in_specs=[
            pl.BlockSpec((TCB, din), lambda j: (j, 0)),
            pl.BlockSpec((din, h), lambda j: (0, 0)),
            pl.BlockSpec((TCB, NC), lambda j: (j, 0)),
        ],
        out_specs=pl.BlockSpec((TCB, h), lambda j: (j, 0)),
        out_shape=jax.ShapeDtypeStruct((n_pad, h), jnp.float32),
    )(x_pad, w, deg_t)


def _final_body(a_ref, xs_ref, d_ref, o_ref):
    dtot = d_ref[:, 0:1] + d_ref[:, 1:2] + 1.0
    dinv = lax.rsqrt(dtot)
    o_ref[...] = (a_ref[0] + a_ref[1] + xs_ref[...]) * dinv


def _tc_final(acc, xs, deg_t):
    n_pad, h = xs.shape
    return pl.pallas_call(
        _final_body,
        grid=(n_pad // TCB,),
        in_specs=[
            pl.BlockSpec((NC, TCB, h), lambda j: (0, j, 0)),
            pl.BlockSpec((TCB, h), lambda j: (j, 0)),
            pl.BlockSpec((TCB, NC), lambda j: (j, 0)),
        ],
        out_specs=pl.BlockSpec((TCB, h), lambda j: (j, 0)),
        out_shape=jax.ShapeDtypeStruct((n_pad, h), jnp.float32),
    )(acc, xs, deg_t)


def kernel(x, edge_index, W):
    n, din = x.shape
    h = W.shape[1]
    e = edge_index.shape[1]

    n_pad = _round_up(n + 1, NS * CHUNK)
    e_per = _round_up(-(-e // NW), 2 * CHUNK)
    e_pad = e_per * NW
    nch = e_per // CHUNK

    src = edge_index[0].astype(jnp.int32)
    dst = edge_index[1].astype(jnp.int32)
    pad = jnp.full((e_pad - e,), n, jnp.int32)
    src3 = jnp.concatenate([src, pad]).reshape(NW, nch, CHUNK)
    dst3 = jnp.concatenate([dst, pad]).reshape(NW, nch, CHUNK)
    x_pad = jnp.concatenate([x, jnp.zeros((n_pad - n, din), jnp.float32)])

    deg = _sc_degree(dst3, n_pad)
    deg_t = jnp.transpose(deg.reshape(NC, n_pad))
    xs = _tc_transform(x_pad, W, deg_t)
    acc = _sc_aggregate(xs, src3, dst3, n_pad)
    out_pad = _tc_final(acc.reshape(NC, n_pad, h), xs, deg_t)
    return out_pad[:n]


# speedup vs baseline: 43.2197x; 3.7725x over previous
import functools

import jax
import jax.numpy as jnp
from jax import lax
from jax.experimental import pallas as pl
from jax.experimental.pallas import tpu as pltpu
from jax.experimental.pallas import tpu_sc as plsc

NC = 2
NS = 16
NW = NC * NS
L = 16
CHUNK = 128
TCB = 2000


def _round_up(v, m):
    return (v + m - 1) // m * m


def _sc_degree(dst3, n_pad):
    nch = dst3.shape[1]
    per = n_pad // NS
    mesh = plsc.VectorSubcoreMesh(core_axis_name="c", subcore_axis_name="s")

    @functools.partial(
        pl.kernel,
        out_type=jax.ShapeDtypeStruct((NC, NS, per), jnp.float32),
        mesh=mesh,
        scratch_types=[
            pltpu.VMEM((nch, CHUNK), jnp.int32),
            pltpu.VMEM((CHUNK,), jnp.float32),
            pltpu.VMEM((_round_up(per, L),), jnp.float32),
            pltpu.VMEM_SHARED((n_pad,), jnp.float32),
        ],
    )
    def deg_kernel(dst_hbm, out_hbm, idx_v, ones_v, zeros_v, deg_s):
        c = lax.axis_index("c")
        s = lax.axis_index("s")
        wid = c * NS + s
        pltpu.sync_copy(dst_hbm.at[wid], idx_v)
        for k in range(CHUNK // L):
            ones_v[pl.ds(k * L, L)] = jnp.ones((L,), jnp.float32)
        for k in range(_round_up(per, L) // L):
            zeros_v[pl.ds(k * L, L)] = jnp.zeros((L,), jnp.float32)
        pltpu.sync_copy(zeros_v.at[pl.ds(0, per)], deg_s.at[pl.ds(s * per, per)])
        plsc.subcore_barrier()

        def body(j, carry):
            pltpu.sync_copy(ones_v, deg_s.at[idx_v.at[j]], add=True)
            return carry

        lax.fori_loop(0, nch, body, 0)
        plsc.subcore_barrier()
        pltpu.sync_copy(deg_s.at[pl.ds(s * per, per)], out_hbm.at[c, s])

    return deg_kernel(dst3)


def _sc_aggregate(xs_pad, src3, dst3, n_pad):
    h = xs_pad.shape[1]
    nch = src3.shape[1]
    per = n_pad // NS
    mesh = plsc.VectorSubcoreMesh(core_axis_name="c", subcore_axis_name="s")

    @functools.partial(
        pl.kernel,
        out_type=jax.ShapeDtypeStruct((NW, per, h), jnp.float32),
        mesh=mesh,
        scratch_types=[
            pltpu.VMEM((CHUNK, h), jnp.float32),
            pltpu.VMEM((CHUNK, h), jnp.float32),
            pltpu.VMEM((CHUNK, h), jnp.float32),
            pltpu.VMEM((CHUNK,), jnp.int32),
            pltpu.VMEM((CHUNK,), jnp.int32),
            pltpu.VMEM((CHUNK,), jnp.int32),
            pltpu.VMEM((CHUNK,), jnp.int32),
            pltpu.VMEM((CHUNK,), jnp.int32),
            pltpu.VMEM((CHUNK,), jnp.int32),
            pltpu.VMEM_SHARED((n_pad, h), jnp.float32),
            pltpu.SemaphoreType.DMA,
            pltpu.SemaphoreType.DMA,
            pltpu.SemaphoreType.DMA,
            pltpu.SemaphoreType.DMA,
            pltpu.SemaphoreType.DMA,
            pltpu.SemaphoreType.DMA,
            pltpu.SemaphoreType.DMA,
            pltpu.SemaphoreType.DMA,
            pltpu.SemaphoreType.DMA,
            pltpu.SemaphoreType.DMA,
            pltpu.SemaphoreType.DMA,
            pltpu.SemaphoreType.DMA,
        ],
    )
    def agg_kernel(xs_hbm, src_hbm, dst_hbm, out_hbm,
                   r0, r1, r2, si0, si1, si2, di0, di1, di2, acc_s,
                   g0, g1, g2, c0, c1, c2, is0, is1, is2, id0, id1, id2):
        c = lax.axis_index("c")
        s = lax.axis_index("s")
        wid = c * NS + s
        rbuf = (r0, r1, r2)
        sibuf = (si0, si1, si2)
        dibuf = (di0, di1, di2)
        gsem = (g0, g1, g2)
        csem = (c0, c1, c2)
        isem = (is0, is1, is2)
        dsem = (id0, id1, id2)

        def start_idx(k, b):
            pltpu.async_copy(src_hbm.at[wid, k], sibuf[b], isem[b])
            pltpu.async_copy(dst_hbm.at[wid, k], dibuf[b], dsem[b])

        def wait_idx(b):
            pltpu.make_async_copy(src_hbm.at[wid, 0], sibuf[b], isem[b]).wait()
            pltpu.make_async_copy(dst_hbm.at[wid, 0], dibuf[b], dsem[b]).wait()

        def start_gather(b):
            pltpu.async_copy(xs_hbm.at[sibuf[b]], rbuf[b], gsem[b])

        def wait_gather(b):
            pltpu.make_async_copy(xs_hbm.at[sibuf[b]], rbuf[b], gsem[b]).wait()

        def start_scatter(b):
            pltpu.async_copy(rbuf[b], acc_s.at[dibuf[b]], csem[b], add=True)

        def wait_scatter(b):
            pltpu.make_async_copy(rbuf[b], acc_s.at[dibuf[b]], csem[b]).wait()

        zero = jnp.zeros((L,), jnp.float32)

        def zrow(i, carry):
            for k in range(h // L):
                r0[i, pl.ds(k * L, L)] = zero
            return carry

        lax.fori_loop(0, CHUNK, zrow, 0)
        nfull = per // CHUNK
        for b in range(nfull):
            pltpu.sync_copy(r0, acc_s.at[pl.ds(s * per + b * CHUNK, CHUNK)])
        rem = per - nfull * CHUNK
        if rem:
            pltpu.sync_copy(r0.at[pl.ds(0, rem)],
                            acc_s.at[pl.ds(s * per + nfull * CHUNK, rem)])
        plsc.subcore_barrier()

        start_idx(0, 0)
        start_idx(1, 1)
        wait_idx(0)
        start_gather(0)
        wait_idx(1)
        start_gather(1)
        wait_gather(0)
        start_scatter(0)
        start_idx(2, 2)

        def step(k, b):
            bp = (b + 2) % 3
            bn = (b + 1) % 3
            wait_idx(bn)
            start_gather(bn)
            wait_gather(b)
            start_scatter(b)
            wait_scatter(bp)
            start_idx(k + 2, bp)

        def trip(i, carry):
            k = 3 * i + 1
            step(k, 1)
            step(k + 1, 2)
            step(k + 2, 0)
            return carry

        lax.fori_loop(0, (nch - 3) // 3, trip, 0)
        b = (nch - 2) % 3
        bn = (b + 1) % 3
        bp = (b + 2) % 3
        wait_idx(bn)
        start_gather(bn)
        wait_gather(b)
        start_scatter(b)
        wait_scatter(bp)
        wait_gather(bn)
        start_scatter(bn)
        wait_scatter(b)
        wait_scatter(bn)

        plsc.subcore_barrier()

        pltpu.sync_copy(acc_s.at[pl.ds(s * per, per)], out_hbm.at[wid])

    return agg_kernel(xs_pad, src3, dst3)


def _transform_body(x_ref, w_ref, d_ref, o_ref):
    dtot = d_ref[:, 0:1] + d_ref[:, 1:2] + 1.0
    dinv = lax.rsqrt(dtot)
    xw = jnp.dot(x_ref[...], w_ref[...], preferred_element_type=jnp.float32)
    o_ref[...] = xw * dinv


def _tc_transform(x, w, deg_t, n_pad):
    n, din = x.shape
    h = w.shape[1]
    return pl.pallas_call(
        _transform_body,
        grid=(n // TCB,),
        in_specs=[
            pl.BlockSpec((TCB, din), lambda j: (j, 0)),
            pl.BlockSpec((din, h), lambda j: (0, 0)),
            pl.BlockSpec((TCB, NC), lambda j: (j, 0)),
        ],
        out_specs=pl.BlockSpec((TCB, h), lambda j: (j, 0)),
        out_shape=jax.ShapeDtypeStruct((n_pad, h), jnp.float32),
    )(x, w, deg_t)


def _final_body(a0_ref, a1_ref, xs_ref, d_ref, o_ref):
    dtot = d_ref[:, 0:1] + d_ref[:, 1:2] + 1.0
    dinv = lax.rsqrt(dtot)
    o_ref[...] = (a0_ref[...] + a1_ref[...] + xs_ref[...]) * dinv


def _tc_final(acc_flat, xs, deg_t, n_pad):
    h = xs.shape[1]
    nblk = 8
    blk = n_pad // nblk
    return pl.pallas_call(
        _final_body,
        grid=(nblk,),
        in_specs=[
            pl.BlockSpec((blk, h), lambda j: (j, 0)),
            pl.BlockSpec((blk, h), lambda j: (j + nblk, 0)),
            pl.BlockSpec((blk, h), lambda j: (j, 0)),
            pl.BlockSpec((blk, NC), lambda j: (j, 0)),
        ],
        out_specs=pl.BlockSpec((blk, h), lambda j: (j, 0)),
        out_shape=jax.ShapeDtypeStruct((n_pad, h), jnp.float32),
    )(acc_flat, acc_flat, xs, deg_t)


def kernel(x, edge_index, W):
    n, din = x.shape
    h = W.shape[1]
    e = edge_index.shape[1]

    n_pad = _round_up(n + 1, NS * 8)
    n_deg = _round_up(n + 1, NS * CHUNK)
    e_per = _round_up(-(-e // NW), 3 * CHUNK)
    e_pad = e_per * NW
    nch = e_per // CHUNK

    src = edge_index[0].astype(jnp.int32)
    dst = edge_index[1].astype(jnp.int32)
    pad = n + jnp.arange(e_pad - e, dtype=jnp.int32) % (n_pad - n)
    src3 = jnp.concatenate([src, pad]).reshape(NW, nch, CHUNK)
    dst3 = jnp.concatenate([dst, pad]).reshape(NW, nch, CHUNK)

    deg = _sc_degree(dst3, n_deg)
    deg_t = jnp.transpose(deg.reshape(NC, n_deg))
    xs = _tc_transform(x, W, deg_t[:n], n_pad)
    acc = _sc_aggregate(xs, src3, dst3, n_pad)
    out_pad = _tc_final(acc.reshape(NC * n_pad, h), xs, deg_t[:n_pad], n_pad)
    return out_pad[:n]
